# baseline (speedup 1.0000x reference)
import functools

import jax
import jax.numpy as jnp
import numpy as np
from jax import lax
from jax.experimental import pallas as pl
from jax.experimental.pallas import tpu as pltpu
from jax.experimental.pallas import tpu_sc as plsc

_VOCAB = 100000
_D = 1280
_E = 128
_B = 16384

_NC = 2
_NS = 16
_NW = _NC * _NS
_BPW = _B // _NW
_CHUNK = 32
_NCHUNK = _BPW // _CHUNK
_HD = _D // 2

_PERM = np.empty(32, dtype=np.int32)
_PERM[0::2] = np.arange(16)
_PERM[1::2] = 16 + np.arange(16)
_COL_IDX = (np.arange(_D) // 32) * 32 + _PERM[np.arange(_D) % 32]


def _sc_gather_bf16(table, idx3):
    mesh = plsc.VectorSubcoreMesh(core_axis_name="c", subcore_axis_name="s")

    @functools.partial(
        pl.kernel,
        mesh=mesh,
        out_type=jax.ShapeDtypeStruct((_B, _HD), jnp.int32),
        scratch_types=[
            pltpu.VMEM((_NCHUNK, _CHUNK), jnp.int32),
            pltpu.VMEM((_CHUNK, _D), jnp.int32),
            pltpu.VMEM((_CHUNK, _D), jnp.int32),
            pltpu.VMEM((_CHUNK, _HD), jnp.int32),
            pltpu.VMEM((_CHUNK, _HD), jnp.int32),
            pltpu.SemaphoreType.DMA,
            pltpu.SemaphoreType.DMA,
            pltpu.SemaphoreType.DMA,
            pltpu.SemaphoreType.DMA,
        ],
    )
    def gather_kernel(table_hbm, idx_hbm, out_hbm, idx_v, f0, f1, p0, p1,
                      gs0, gs1, ws0, ws1):
        wid = lax.axis_index("s") * _NC + lax.axis_index("c")
        base = wid * _BPW
        pltpu.sync_copy(idx_hbm.at[wid], idx_v)

        fbufs = (f0, f1)
        pbufs = (p0, p1)
        gsems = (gs0, gs1)
        wsems = (ws0, ws1)

        def start_gather(j, buf, sem):
            return pltpu.async_copy(table_hbm.at[idx_v.at[j]], buf, sem)

        def start_write(j, buf, sem):
            return pltpu.async_copy(
                buf, out_hbm.at[pl.ds(base + j * _CHUNK, _CHUNK)], sem)

        def convert(fbuf, pbuf):
            def row(r, carry):
                def kblk(kk, carry2):
                    for u in range(8):
                        off = 256 * kk + 32 * u
                        a = fbuf[r, pl.ds(off, 16)]
                        c = fbuf[r, pl.ds(off + 16, 16)]
                        lo = lax.shift_right_logical(a + 0x8000, 16)
                        hi = (c + 0x8000) & jnp.int32(-65536)
                        pbuf[r, pl.ds(128 * kk + 16 * u, 16)] = lo | hi
                    return carry2
                lax.fori_loop(0, _D // 256, kblk, 0)
                return carry
            lax.fori_loop(0, _CHUNK, row, 0)

        hg = [None, None]
        hw = [None, None]
        hg[0] = start_gather(0, fbufs[0], gsems[0])
        for j in range(_NCHUNK):
            cur = j % 2
            nxt = 1 - cur
            if j + 1 < _NCHUNK:
                hg[nxt] = start_gather(j + 1, fbufs[nxt], gsems[nxt])
            hg[cur].wait()
            if j >= 2:
                hw[cur].wait()
            convert(fbufs[cur], pbufs[cur])
            hw[cur] = start_write(j, pbufs[cur], wsems[cur])
        hw[(_NCHUNK - 2) % 2].wait()
        hw[(_NCHUNK - 1) % 2].wait()

    return gather_kernel(table, idx3)


_RB = 1024


def _head_body(emb_ref, w_ref, b_ref, g_ref, beta_ref, out_ref):
    y = jnp.dot(emb_ref[...], w_ref[...], preferred_element_type=jnp.float32)
    y = y + b_ref[...]
    mu = jnp.mean(y, axis=-1, keepdims=True)
    var = jnp.mean(jnp.square(y - mu), axis=-1, keepdims=True)
    out_ref[...] = (y - mu) * lax.rsqrt(var + 1e-5) * g_ref[...] + beta_ref[...]


def _tc_head(emb, W, b2, g2, beta2):
    grid = (_B // _RB,)
    return pl.pallas_call(
        _head_body,
        grid=grid,
        in_specs=[
            pl.BlockSpec((_RB, _D), lambda i: (i, 0)),
            pl.BlockSpec((_D, _E), lambda i: (0, 0)),
            pl.BlockSpec((1, _E), lambda i: (0, 0)),
            pl.BlockSpec((1, _E), lambda i: (0, 0)),
            pl.BlockSpec((1, _E), lambda i: (0, 0)),
        ],
        out_specs=pl.BlockSpec((_RB, _E), lambda i: (i, 0)),
        out_shape=jax.ShapeDtypeStruct((_B, _E), jnp.float32),
        compiler_params=pltpu.CompilerParams(
            dimension_semantics=("arbitrary",),
        ),
    )(emb, W, b2, g2, beta2)


def kernel(indices, table, W, b, gamma, beta):
    idx3 = indices.astype(jnp.int32).reshape(_NW, _NCHUNK, _CHUNK)
    table_i32 = lax.bitcast_convert_type(table, jnp.int32)
    packed = _sc_gather_bf16(table_i32, idx3)
    emb = lax.bitcast_convert_type(packed, jnp.bfloat16).reshape(_B, _D)
    Wp = jnp.take(W, jnp.asarray(_COL_IDX), axis=0).astype(jnp.bfloat16)
    out = _tc_head(emb, Wp, b.reshape(1, _E), gamma.reshape(1, _E),
                   beta.reshape(1, _E))
    return out


# speedup vs baseline: 1.0445x; 1.0445x over previous
import functools

import jax
import jax.numpy as jnp
import numpy as np
from jax import lax
from jax.experimental import pallas as pl
from jax.experimental.pallas import tpu as pltpu
from jax.experimental.pallas import tpu_sc as plsc

_VOCAB = 100000
_D = 1280
_E = 128
_B = 16384

_NC = 2
_NS = 16
_NW = _NC * _NS
_BPW = _B // _NW
_CHUNK = 32
_NCHUNK = _BPW // _CHUNK
_HD = _D // 2

_PERM = np.empty(32, dtype=np.int32)
_PERM[0::2] = np.arange(16)
_PERM[1::2] = 16 + np.arange(16)
_COL_IDX = (np.arange(_D) // 32) * 32 + _PERM[np.arange(_D) % 32]


def _sc_gather_bf16(table, idx3):
    mesh = plsc.VectorSubcoreMesh(core_axis_name="c", subcore_axis_name="s")

    @functools.partial(
        pl.kernel,
        mesh=mesh,
        out_type=jax.ShapeDtypeStruct((_B * _HD,), jnp.int32),
        scratch_types=[
            pltpu.VMEM((_NCHUNK, _CHUNK), jnp.int32),
            pltpu.VMEM((_CHUNK, _D), jnp.int32),
            pltpu.VMEM((_CHUNK, _D), jnp.int32),
            pltpu.VMEM((_CHUNK * _HD,), jnp.int32),
            pltpu.VMEM((_CHUNK * _HD,), jnp.int32),
            pltpu.SemaphoreType.DMA,
            pltpu.SemaphoreType.DMA,
            pltpu.SemaphoreType.DMA,
            pltpu.SemaphoreType.DMA,
        ],
    )
    def gather_kernel(table_hbm, idx_hbm, out_hbm, idx_v, f0, f1, p0, p1,
                      gs0, gs1, ws0, ws1):
        wid = lax.axis_index("s") * _NC + lax.axis_index("c")
        base = wid * _BPW
        pltpu.sync_copy(idx_hbm.at[wid], idx_v)

        fbufs = (f0, f1)
        pbufs = (p0, p1)
        gsems = (gs0, gs1)
        wsems = (ws0, ws1)

        def start_gather(j, buf, sem):
            return pltpu.async_copy(table_hbm.at[idx_v.at[j]], buf, sem)

        def start_write(j, buf, sem):
            return pltpu.async_copy(
                buf,
                out_hbm.at[pl.ds((base + j * _CHUNK) * _HD, _CHUNK * _HD)],
                sem)

        def convert(fbuf, pbuf):
            ones = lax.iota(jnp.int32, 16) >= 0

            @plsc.parallel_loop(0, _CHUNK * 4, unroll=2)
            def body(i):
                r = lax.shift_right_logical(i, 2)
                kk = i & 3
                pb = r * _HD + kk * (_HD // 4)
                fb = kk * (_D // 4)
                for u in range(_D // 128):
                    a = fbuf[r, pl.ds(fb + 32 * u, 16)]
                    c = fbuf[r, pl.ds(fb + 32 * u + 16, 16)]
                    lo = lax.shift_right_logical(a + 0x8000, 16)
                    hi = (c + 0x8000) & jnp.int32(-65536)
                    pbuf[pl.ds(pb + 16 * u, 16)] = lo | hi

        hg = [None, None]
        hw = [None, None]
        hg[0] = start_gather(0, fbufs[0], gsems[0])
        for j in range(_NCHUNK):
            cur = j % 2
            nxt = 1 - cur
            if j + 1 < _NCHUNK:
                hg[nxt] = start_gather(j + 1, fbufs[nxt], gsems[nxt])
            hg[cur].wait()
            if j >= 2:
                hw[cur].wait()
            convert(fbufs[cur], pbufs[cur])
            hw[cur] = start_write(j, pbufs[cur], wsems[cur])
        hw[(_NCHUNK - 2) % 2].wait()
        hw[(_NCHUNK - 1) % 2].wait()

    return gather_kernel(table, idx3)


_RB = 1024


def _head_body(emb_ref, w_ref, b_ref, g_ref, beta_ref, out_ref):
    y = jnp.dot(emb_ref[...], w_ref[...], preferred_element_type=jnp.float32)
    y = y + b_ref[...]
    mu = jnp.mean(y, axis=-1, keepdims=True)
    var = jnp.mean(jnp.square(y - mu), axis=-1, keepdims=True)
    out_ref[...] = (y - mu) * lax.rsqrt(var + 1e-5) * g_ref[...] + beta_ref[...]


def _tc_head(emb, W, b2, g2, beta2):
    grid = (_B // _RB,)
    return pl.pallas_call(
        _head_body,
        grid=grid,
        in_specs=[
            pl.BlockSpec((_RB, _D), lambda i: (i, 0)),
            pl.BlockSpec((_D, _E), lambda i: (0, 0)),
            pl.BlockSpec((1, _E), lambda i: (0, 0)),
            pl.BlockSpec((1, _E), lambda i: (0, 0)),
            pl.BlockSpec((1, _E), lambda i: (0, 0)),
        ],
        out_specs=pl.BlockSpec((_RB, _E), lambda i: (i, 0)),
        out_shape=jax.ShapeDtypeStruct((_B, _E), jnp.float32),
        compiler_params=pltpu.CompilerParams(
            dimension_semantics=("arbitrary",),
        ),
    )(emb, W, b2, g2, beta2)


def kernel(indices, table, W, b, gamma, beta):
    idx3 = indices.astype(jnp.int32).reshape(_NW, _NCHUNK, _CHUNK)
    table_i32 = lax.bitcast_convert_type(table, jnp.int32)
    packed = _sc_gather_bf16(table_i32, idx3).reshape(_B, _HD)
    emb = lax.bitcast_convert_type(packed, jnp.bfloat16).reshape(_B, _D)
    Wp = jnp.take(W, jnp.asarray(_COL_IDX), axis=0).astype(jnp.bfloat16)
    out = _tc_head(emb, Wp, b.reshape(1, _E), gamma.reshape(1, _E),
                   beta.reshape(1, _E))
    return out


# speedup vs baseline: 7.5592x; 7.2373x over previous
import functools

import jax
import jax.numpy as jnp
import numpy as np
from jax import lax
from jax.experimental import pallas as pl
from jax.experimental.pallas import tpu as pltpu
from jax.experimental.pallas import tpu_sc as plsc

_VOCAB = 100000
_D = 1280
_E = 128
_B = 16384

_NC = 2
_NS = 16
_NW = _NC * _NS
_BPW = _B // _NW
_CHUNK = 32
_NCHUNK = _BPW // _CHUNK
_HD = _D // 2

_QS = np.arange(_D // 2)
_GE = ((_QS // 16) * 32 + _QS % 16).astype(np.int32)


def _sc_gather_bf16(table, idx3):
    mesh = plsc.VectorSubcoreMesh(core_axis_name="c", subcore_axis_name="s")

    @functools.partial(
        pl.kernel,
        mesh=mesh,
        out_type=jax.ShapeDtypeStruct((_B, _HD), jnp.int32),
        scratch_types=[
            pltpu.VMEM((_NCHUNK, _CHUNK), jnp.int32),
            pltpu.VMEM((_CHUNK, _D), jnp.float32),
            pltpu.VMEM((_CHUNK, _D), jnp.float32),
            pltpu.VMEM((_CHUNK, _HD), jnp.int32),
            pltpu.VMEM((_CHUNK, _HD), jnp.int32),
            pltpu.SemaphoreType.DMA,
            pltpu.SemaphoreType.DMA,
            pltpu.SemaphoreType.DMA,
            pltpu.SemaphoreType.DMA,
        ],
    )
    def gather_kernel(table_hbm, idx_hbm, out_hbm, idx_v, f0, f1, p0, p1,
                      gs0, gs1, ws0, ws1):
        wid = lax.axis_index("s") * _NC + lax.axis_index("c")
        base = wid * _BPW
        pltpu.sync_copy(idx_hbm.at[wid], idx_v)

        fbufs = (f0, f1)
        pbufs = (p0, p1)
        gsems = (gs0, gs1)
        wsems = (ws0, ws1)

        def start_gather(j, buf, sem):
            return pltpu.async_copy(table_hbm.at[idx_v.at[j]], buf, sem)

        def start_write(j, buf, sem):
            return pltpu.async_copy(
                buf, out_hbm.at[pl.ds(base + j * _CHUNK, _CHUNK)], sem)

        def convert(fbuf, pbuf):
            ones = lax.iota(jnp.int32, 16) >= 0

            @plsc.parallel_loop(0, _CHUNK * 4, unroll=2)
            def body(i):
                r = lax.shift_right_logical(i, 2)
                kk = i & 3
                pb = kk * (_HD // 4)
                fb = kk * (_D // 4)
                for u in range(_D // 128):
                    a = lax.bitcast_convert_type(
                        fbuf[r, pl.ds(fb + 32 * u, 16)], jnp.int32)
                    c = lax.bitcast_convert_type(
                        fbuf[r, pl.ds(fb + 32 * u + 16, 16)], jnp.int32)
                    lo = lax.shift_right_logical(a + 0x8000, 16)
                    hi = (c + 0x8000) & jnp.int32(-65536)
                    pbuf[r, pl.ds(pb + 16 * u, 16)] = lo | hi

        hg = [None, None]
        hw = [None, None]
        hg[0] = start_gather(0, fbufs[0], gsems[0])
        for j in range(_NCHUNK):
            cur = j % 2
            nxt = 1 - cur
            if j + 1 < _NCHUNK:
                hg[nxt] = start_gather(j + 1, fbufs[nxt], gsems[nxt])
            hg[cur].wait()
            if j >= 2:
                hw[cur].wait()
            convert(fbufs[cur], pbufs[cur])
            hw[cur] = start_write(j, pbufs[cur], wsems[cur])
        hw[(_NCHUNK - 2) % 2].wait()
        hw[(_NCHUNK - 1) % 2].wait()

    return gather_kernel(table, idx3)


_RB = 1024


def _head_body(p_ref, we_ref, wo_ref, b_ref, g_ref, beta_ref, out_ref):
    w = p_ref[...]
    xe = lax.bitcast_convert_type(w << 16, jnp.float32)
    xo = lax.bitcast_convert_type(w & jnp.int32(-65536), jnp.float32)
    y = jnp.dot(xe, we_ref[...], preferred_element_type=jnp.float32)
    y = y + jnp.dot(xo, wo_ref[...], preferred_element_type=jnp.float32)
    y = y + b_ref[...]
    mu = jnp.mean(y, axis=-1, keepdims=True)
    var = jnp.mean(jnp.square(y - mu), axis=-1, keepdims=True)
    out_ref[...] = (y - mu) * lax.rsqrt(var + 1e-5) * g_ref[...] + beta_ref[...]


def _tc_head(packed, We, Wo, b2, g2, beta2):
    grid = (_B // _RB,)
    return pl.pallas_call(
        _head_body,
        grid=grid,
        in_specs=[
            pl.BlockSpec((_RB, _HD), lambda i: (i, 0)),
            pl.BlockSpec((_HD, _E), lambda i: (0, 0)),
            pl.BlockSpec((_HD, _E), lambda i: (0, 0)),
            pl.BlockSpec((1, _E), lambda i: (0, 0)),
            pl.BlockSpec((1, _E), lambda i: (0, 0)),
            pl.BlockSpec((1, _E), lambda i: (0, 0)),
        ],
        out_specs=pl.BlockSpec((_RB, _E), lambda i: (i, 0)),
        out_shape=jax.ShapeDtypeStruct((_B, _E), jnp.float32),
        compiler_params=pltpu.CompilerParams(
            dimension_semantics=("arbitrary",),
        ),
    )(packed, We, Wo, b2, g2, beta2)


def kernel(indices, table, W, b, gamma, beta):
    idx3 = indices.astype(jnp.int32).reshape(_NW, _NCHUNK, _CHUNK)
    packed = _sc_gather_bf16(table, idx3)
    ge = jnp.asarray(_GE)
    We = jnp.take(W, ge, axis=0)
    Wo = jnp.take(W, ge + 16, axis=0)
    out = _tc_head(packed, We, Wo, b.reshape(1, _E), gamma.reshape(1, _E),
                   beta.reshape(1, _E))
    return out


# speedup vs baseline: 7.9001x; 1.0451x over previous
import functools

import jax
import jax.numpy as jnp
import numpy as np
from jax import lax
from jax.experimental import pallas as pl
from jax.experimental.pallas import tpu as pltpu
from jax.experimental.pallas import tpu_sc as plsc

_VOCAB = 100000
_D = 1280
_E = 128
_B = 16384

_NC = 2
_NS = 16
_NW = _NC * _NS
_BPW = _B // _NW
_CHUNK = 32
_NCHUNK = _BPW // _CHUNK
_HD = _D // 2

_QS = np.arange(_D // 2)
_GE = ((_QS // 16) * 32 + _QS % 16).astype(np.int32)


def _sc_gather_bf16(table, idx3):
    mesh = plsc.VectorSubcoreMesh(core_axis_name="c", subcore_axis_name="s")

    @functools.partial(
        pl.kernel,
        mesh=mesh,
        out_type=jax.ShapeDtypeStruct((_B, _HD), jnp.int32),
        scratch_types=[
            pltpu.VMEM((_NCHUNK, _CHUNK), jnp.int32),
            pltpu.VMEM((_CHUNK, _D), jnp.float32),
            pltpu.VMEM((_CHUNK, _D), jnp.float32),
            pltpu.VMEM((_CHUNK, _HD), jnp.int32),
            pltpu.VMEM((_CHUNK, _HD), jnp.int32),
            pltpu.SemaphoreType.DMA,
            pltpu.SemaphoreType.DMA,
            pltpu.SemaphoreType.DMA,
            pltpu.SemaphoreType.DMA,
        ],
    )
    def gather_kernel(table_hbm, idx_hbm, out_hbm, idx_v, f0, f1, p0, p1,
                      gs0, gs1, ws0, ws1):
        wid = lax.axis_index("s") * _NC + lax.axis_index("c")
        base = wid * _BPW
        pltpu.sync_copy(idx_hbm.at[wid], idx_v)

        fbufs = (f0, f1)
        pbufs = (p0, p1)
        gsems = (gs0, gs1)
        wsems = (ws0, ws1)

        def start_gather(j, buf, sem):
            return pltpu.async_copy(table_hbm.at[idx_v.at[j]], buf, sem)

        def start_write(j, buf, sem):
            return pltpu.async_copy(
                buf, out_hbm.at[pl.ds(base + j * _CHUNK, _CHUNK)], sem)

        def convert(fbuf, pbuf):
            ones = lax.iota(jnp.int32, 16) >= 0

            @plsc.parallel_loop(0, _CHUNK * 4, unroll=2)
            def body(i):
                r = lax.shift_right_logical(i, 2)
                kk = i & 3
                pb = kk * (_HD // 4)
                fb = kk * (_D // 4)
                for u in range(_D // 128):
                    a = lax.bitcast_convert_type(
                        fbuf[r, pl.ds(fb + 32 * u, 16)], jnp.int32)
                    c = lax.bitcast_convert_type(
                        fbuf[r, pl.ds(fb + 32 * u + 16, 16)], jnp.int32)
                    lo = lax.shift_right_logical(a + 0x8000, 16)
                    hi = (c + 0x8000) & jnp.int32(-65536)
                    pbuf[r, pl.ds(pb + 16 * u, 16)] = lo | hi


        hg = [None, None]
        hw = [None, None]
        hg[0] = start_gather(0, fbufs[0], gsems[0])
        for j in range(_NCHUNK):
            cur = j % 2
            nxt = 1 - cur
            if j + 1 < _NCHUNK:
                hg[nxt] = start_gather(j + 1, fbufs[nxt], gsems[nxt])
            hg[cur].wait()
            if j >= 2:
                hw[cur].wait()
            convert(fbufs[cur], pbufs[cur])
            hw[cur] = start_write(j, pbufs[cur], wsems[cur])
        hw[(_NCHUNK - 2) % 2].wait()
        hw[(_NCHUNK - 1) % 2].wait()

    return gather_kernel(table, idx3)


_RB = 2048


def _head_body(p_ref, we_ref, wo_ref, b_ref, g_ref, beta_ref, out_ref):
    w = p_ref[...]
    xe = lax.bitcast_convert_type(w << 16, jnp.float32)
    xo = lax.bitcast_convert_type(w & jnp.int32(-65536), jnp.float32)
    y = jnp.dot(xe, we_ref[...], preferred_element_type=jnp.float32)
    y = y + jnp.dot(xo, wo_ref[...], preferred_element_type=jnp.float32)
    y = y + b_ref[...]
    mu = jnp.mean(y, axis=-1, keepdims=True)
    var = jnp.mean(jnp.square(y - mu), axis=-1, keepdims=True)
    out_ref[...] = (y - mu) * lax.rsqrt(var + 1e-5) * g_ref[...] + beta_ref[...]


def _tc_head(packed, We, Wo, b2, g2, beta2):
    grid = (_B // _RB,)
    return pl.pallas_call(
        _head_body,
        grid=grid,
        in_specs=[
            pl.BlockSpec((_RB, _HD), lambda i: (i, 0)),
            pl.BlockSpec((_HD, _E), lambda i: (0, 0)),
            pl.BlockSpec((_HD, _E), lambda i: (0, 0)),
            pl.BlockSpec((1, _E), lambda i: (0, 0)),
            pl.BlockSpec((1, _E), lambda i: (0, 0)),
            pl.BlockSpec((1, _E), lambda i: (0, 0)),
        ],
        out_specs=pl.BlockSpec((_RB, _E), lambda i: (i, 0)),
        out_shape=jax.ShapeDtypeStruct((_B, _E), jnp.float32),
        compiler_params=pltpu.CompilerParams(
            dimension_semantics=("arbitrary",),
        ),
    )(packed, We, Wo, b2, g2, beta2)


def kernel(indices, table, W, b, gamma, beta):
    idx3 = indices.astype(jnp.int32).reshape(_NW, _NCHUNK, _CHUNK)
    packed = _sc_gather_bf16(table, idx3)
    ge = jnp.asarray(_GE)
    We = jnp.take(W, ge, axis=0)
    Wo = jnp.take(W, ge + 16, axis=0)
    out = _tc_head(packed, We, Wo, b.reshape(1, _E), gamma.reshape(1, _E),
                   beta.reshape(1, _E))
    return out


# speedup vs baseline: 7.9779x; 1.0098x over previous
import functools

import jax
import jax.numpy as jnp
import numpy as np
from jax import lax
from jax.experimental import pallas as pl
from jax.experimental.pallas import tpu as pltpu
from jax.experimental.pallas import tpu_sc as plsc

_VOCAB = 100000
_D = 1280
_E = 128
_B = 16384

_NC = 2
_NS = 16
_NW = _NC * _NS
_BPW = _B // _NW
_CHUNK = 32
_NCHUNK = _BPW // _CHUNK
_HD = _D // 2

_QS = np.arange(_D // 2)
_GE = ((_QS // 16) * 32 + _QS % 16).astype(np.int32)


def _sc_gather_bf16(table, idx3):
    mesh = plsc.VectorSubcoreMesh(core_axis_name="c", subcore_axis_name="s")

    @functools.partial(
        pl.kernel,
        mesh=mesh,
        out_type=jax.ShapeDtypeStruct((_B, _HD), jnp.int32),
        scratch_types=[
            pltpu.VMEM((_NCHUNK, _CHUNK), jnp.int32),
            pltpu.VMEM((_CHUNK, _D), jnp.float32),
            pltpu.VMEM((_CHUNK, _D), jnp.float32),
            pltpu.VMEM((_CHUNK, _HD), jnp.int32),
            pltpu.VMEM((_CHUNK, _HD), jnp.int32),
            pltpu.SemaphoreType.DMA,
            pltpu.SemaphoreType.DMA,
            pltpu.SemaphoreType.DMA,
            pltpu.SemaphoreType.DMA,
        ],
    )
    def gather_kernel(table_hbm, idx_hbm, out_hbm, idx_v, f0, f1, p0, p1,
                      gs0, gs1, ws0, ws1):
        wid = lax.axis_index("s") * _NC + lax.axis_index("c")
        base = wid * _BPW
        pltpu.sync_copy(idx_hbm.at[wid], idx_v)

        fbufs = (f0, f1)
        pbufs = (p0, p1)
        gsems = (gs0, gs1)
        wsems = (ws0, ws1)

        def start_gather(j, buf, sem):
            return pltpu.async_copy(table_hbm.at[idx_v.at[j]], buf, sem)

        def start_write(j, buf, sem):
            return pltpu.async_copy(
                buf, out_hbm.at[pl.ds(base + j * _CHUNK, _CHUNK)], sem)

        def convert(fbuf, pbuf):
            ones = lax.iota(jnp.int32, 16) >= 0

            @plsc.parallel_loop(0, _CHUNK * 4, unroll=2)
            def body(i):
                r = lax.shift_right_logical(i, 2)
                kk = i & 3
                pb = kk * (_HD // 4)
                fb = kk * (_D // 4)
                for u in range(_D // 128):
                    a = lax.bitcast_convert_type(
                        fbuf[r, pl.ds(fb + 32 * u, 16)], jnp.int32)
                    c = lax.bitcast_convert_type(
                        fbuf[r, pl.ds(fb + 32 * u + 16, 16)], jnp.int32)
                    lo = lax.shift_right_logical(a + 0x8000, 16)
                    hi = (c + 0x8000) & jnp.int32(-65536)
                    pbuf[r, pl.ds(pb + 16 * u, 16)] = lo | hi


        hg = [None, None]
        hw = [None, None]
        hg[0] = start_gather(0, fbufs[0], gsems[0])
        for j in range(_NCHUNK):
            cur = j % 2
            nxt = 1 - cur
            if j + 1 < _NCHUNK:
                hg[nxt] = start_gather(j + 1, fbufs[nxt], gsems[nxt])
            hg[cur].wait()
            if j >= 2:
                hw[cur].wait()
            convert(fbufs[cur], pbufs[cur])
            hw[cur] = start_write(j, pbufs[cur], wsems[cur])
        hw[(_NCHUNK - 2) % 2].wait()
        hw[(_NCHUNK - 1) % 2].wait()

    return gather_kernel(table, idx3)


_RB = 4096


def _head_body(p_ref, we_ref, wo_ref, b_ref, g_ref, beta_ref, out_ref):
    w = p_ref[...]
    xe = lax.bitcast_convert_type(w << 16, jnp.float32)
    xo = lax.bitcast_convert_type(w & jnp.int32(-65536), jnp.float32)
    y = jnp.dot(xe, we_ref[...], preferred_element_type=jnp.float32)
    y = y + jnp.dot(xo, wo_ref[...], preferred_element_type=jnp.float32)
    y = y + b_ref[...]
    mu = jnp.mean(y, axis=-1, keepdims=True)
    var = jnp.mean(jnp.square(y - mu), axis=-1, keepdims=True)
    out_ref[...] = (y - mu) * lax.rsqrt(var + 1e-5) * g_ref[...] + beta_ref[...]


def _tc_head(packed, We, Wo, b2, g2, beta2):
    grid = (_B // _RB,)
    return pl.pallas_call(
        _head_body,
        grid=grid,
        in_specs=[
            pl.BlockSpec((_RB, _HD), lambda i: (i, 0)),
            pl.BlockSpec((_HD, _E), lambda i: (0, 0)),
            pl.BlockSpec((_HD, _E), lambda i: (0, 0)),
            pl.BlockSpec((1, _E), lambda i: (0, 0)),
            pl.BlockSpec((1, _E), lambda i: (0, 0)),
            pl.BlockSpec((1, _E), lambda i: (0, 0)),
        ],
        out_specs=pl.BlockSpec((_RB, _E), lambda i: (i, 0)),
        out_shape=jax.ShapeDtypeStruct((_B, _E), jnp.float32),
        compiler_params=pltpu.CompilerParams(
            dimension_semantics=("arbitrary",),
        ),
    )(packed, We, Wo, b2, g2, beta2)


def kernel(indices, table, W, b, gamma, beta):
    idx3 = indices.astype(jnp.int32).reshape(_NW, _NCHUNK, _CHUNK)
    packed = _sc_gather_bf16(table, idx3)
    ge = jnp.asarray(_GE)
    We = jnp.take(W, ge, axis=0)
    Wo = jnp.take(W, ge + 16, axis=0)
    out = _tc_head(packed, We, Wo, b.reshape(1, _E), gamma.reshape(1, _E),
                   beta.reshape(1, _E))
    return out
